# baseline (speedup 1.0000x reference)
import functools

import jax
import jax.numpy as jnp
from jax import lax
from jax.experimental import pallas as pl
from jax.experimental.pallas import tpu as pltpu
from jax.experimental.pallas import tpu_sc as plsc

_NC = 2
_NS = 16
_NW = _NC * _NS
_CHUNK = 128


def _mesh():
    return plsc.VectorSubcoreMesh(core_axis_name="c", subcore_axis_name="s")


def _zero_1d(ref, n):
    def f(i, _):
        ref[pl.ds(i * 16, 16)] = jnp.zeros((16,), jnp.float32)
        return 0
    lax.fori_loop(0, n // 16, f, 0)


def _zero_2d(ref, r, cdim):
    def f(i, _):
        ref[i // (cdim // 16), pl.ds((i % (cdim // 16)) * 16, 16)] = (
            jnp.zeros((16,), jnp.float32))
        return 0
    lax.fori_loop(0, r * (cdim // 16), f, 0)


def _make_deg(E, NPAD, KB, nblk):
    ept = E // _NW
    nch = ept // _CHUNK
    npt = NPAD // _NS

    @functools.partial(
        pl.kernel,
        mesh=_mesh(),
        out_type=[
            jax.ShapeDtypeStruct((_NC * NPAD,), jnp.float32),
            jax.ShapeDtypeStruct((_NC * NPAD,), jnp.float32),
        ],
        scratch_types=[
            pltpu.VMEM((nblk, KB, _CHUNK), jnp.int32),
            pltpu.VMEM((nblk, KB, _CHUNK), jnp.int32),
            pltpu.VMEM((_CHUNK,), jnp.float32),
            pltpu.VMEM((npt,), jnp.float32),
            pltpu.SemaphoreType.DMA,
            pltpu.SemaphoreType.DMA,
            pltpu.VMEM_SHARED((NPAD,), jnp.float32),
            pltpu.VMEM_SHARED((NPAD,), jnp.float32),
        ],
    )
    def deg_kernel(edge_h, dego_h, degi_h, isrc_v, idst_v, ones_v,
                   buf_v, sem_o, sem_i, dego_sp, degi_sp):
        cid = lax.axis_index("c")
        sid = lax.axis_index("s")
        wid = sid * _NC + cid

        pltpu.sync_copy(edge_h.at[0, wid], isrc_v)
        pltpu.sync_copy(edge_h.at[1, wid], idst_v)

        def fill(i, _):
            ones_v[pl.ds(i * 16, 16)] = jnp.ones((16,), jnp.float32)
            return 0
        lax.fori_loop(0, _CHUNK // 16, fill, 0)
        _zero_1d(buf_v, npt)

        pltpu.sync_copy(buf_v, dego_sp.at[pl.ds(sid * npt, npt)])
        pltpu.sync_copy(buf_v, degi_sp.at[pl.ds(sid * npt, npt)])
        plsc.subcore_barrier()

        def blk(b, _):
            def f(i, _):
                k = b * 5 + i
                pltpu.async_copy(ones_v, dego_sp.at[isrc_v.at[k // KB, k % KB]],
                                 sem_o, add=True)
                pltpu.async_copy(ones_v, degi_sp.at[idst_v.at[k // KB, k % KB]],
                                 sem_i, add=True)
                return 0
            lax.fori_loop(0, 5, f, 0)

            def d(i, _):
                k = b * 5 + i
                pltpu.make_async_copy(ones_v,
                                      dego_sp.at[isrc_v.at[k // KB, k % KB]],
                                      sem_o).wait()
                pltpu.make_async_copy(ones_v,
                                      degi_sp.at[idst_v.at[k // KB, k % KB]],
                                      sem_i).wait()
                return 0
            lax.fori_loop(0, 5, d, 0)
            return 0
        lax.fori_loop(0, nch // 5, blk, 0)

        def tail(i, _):
            pltpu.sync_copy(ones_v, dego_sp.at[isrc_v.at[i // KB, i % KB]],
                            add=True)
            pltpu.sync_copy(ones_v, degi_sp.at[idst_v.at[i // KB, i % KB]],
                            add=True)
            return 0
        lax.fori_loop((nch // 5) * 5, nch, tail, 0)
        plsc.subcore_barrier()

        pltpu.sync_copy(dego_sp.at[pl.ds(sid * npt, npt)], buf_v)
        pltpu.sync_copy(buf_v, dego_h.at[pl.ds(cid * NPAD + sid * npt, npt)])
        pltpu.sync_copy(degi_sp.at[pl.ds(sid * npt, npt)], buf_v)
        pltpu.sync_copy(buf_v, degi_h.at[pl.ds(cid * NPAD + sid * npt, npt)])

    return deg_kernel


def _make_mm(NPAD, D, BR=2048):
    def body(x_ref, w1_ref, dgo0, dgo1, dgi0, dgi1, yn_ref, ns_ref, nd_ref):
        ns = lax.rsqrt(jnp.maximum(dgo0[...] + dgo1[...], 1.0))
        nd = lax.rsqrt(jnp.maximum(dgi0[...] + dgi1[...], 1.0))
        ns_ref[...] = ns
        nd_ref[...] = nd
        yn_ref[...] = jnp.dot(x_ref[...], w1_ref[...],
                              preferred_element_type=jnp.float32) * ns[:, None]

    grid = NPAD // BR
    nb = NPAD // BR
    return pl.pallas_call(
        body,
        grid=(grid,),
        in_specs=[
            pl.BlockSpec((BR, D), lambda i: (i, 0)),
            pl.BlockSpec((D, D), lambda i: (0, 0)),
            pl.BlockSpec((BR,), lambda i: (i,)),
            pl.BlockSpec((BR,), lambda i: (i + nb,)),
            pl.BlockSpec((BR,), lambda i: (i,)),
            pl.BlockSpec((BR,), lambda i: (i + nb,)),
        ],
        out_specs=[
            pl.BlockSpec((BR, D), lambda i: (i, 0)),
            pl.BlockSpec((BR,), lambda i: (i,)),
            pl.BlockSpec((BR,), lambda i: (i,)),
        ],
        out_shape=[
            jax.ShapeDtypeStruct((NPAD, D), jnp.float32),
            jax.ShapeDtypeStruct((NPAD,), jnp.float32),
            jax.ShapeDtypeStruct((NPAD,), jnp.float32),
        ],
    )


def _make_prop(E, NPAD, D, KB, nblk):
    ept = E // _NW
    nch = ept // _CHUNK
    npt = NPAD // _NS
    nwo = npt // _CHUNK

    NBUF = 2
    LA = 1

    @functools.partial(
        pl.kernel,
        mesh=_mesh(),
        out_type=[
            jax.ShapeDtypeStruct((_NC, NPAD, D), jnp.float32),
            jax.ShapeDtypeStruct((_NC * NPAD,), jnp.float32),
        ],
        scratch_types=[
            pltpu.VMEM((KB, _CHUNK), jnp.int32),
            pltpu.VMEM((KB, _CHUNK), jnp.int32),
            pltpu.VMEM((NBUF, _CHUNK, D), jnp.float32),
            pltpu.VMEM((NBUF, _CHUNK), jnp.float32),
            pltpu.SemaphoreType.DMA((NBUF,)),
            pltpu.SemaphoreType.DMA((NBUF,)),
            pltpu.SemaphoreType.DMA((NBUF,)),
            pltpu.SemaphoreType.DMA((NBUF,)),
            pltpu.VMEM_SHARED((NPAD, D), jnp.float32),
            pltpu.VMEM_SHARED((NPAD,), jnp.float32),
        ],
    )
    def prop_kernel(edge_h, yn_h, nd_h, agg_h, s_h,
                    isrc_v, idst_v, rows_v, nval_v, sem_r, sem_n,
                    sem_w, sem_x, agg_sp, s_sp):
        cid = lax.axis_index("c")
        sid = lax.axis_index("s")
        wid = sid * _NC + cid

        _zero_2d(rows_v.at[0], _CHUNK, D)
        _zero_1d(nval_v.at[0], _CHUNK)

        def zstep(k, _):
            off = pl.multiple_of(sid * npt + k * _CHUNK, _CHUNK)
            pltpu.sync_copy(rows_v.at[0], agg_sp.at[pl.ds(off, _CHUNK)])
            pltpu.sync_copy(nval_v.at[0], s_sp.at[pl.ds(off, _CHUNK)])
            return 0
        lax.fori_loop(0, nwo, zstep, 0)
        plsc.subcore_barrier()

        def fire_gather(j, b):
            pltpu.async_copy(yn_h.at[isrc_v.at[j]], rows_v.at[b], sem_r.at[b])
            pltpu.async_copy(nd_h.at[idst_v.at[j]], nval_v.at[b], sem_n.at[b])

        def drain_gather(j, b):
            pltpu.make_async_copy(yn_h.at[isrc_v.at[j]], rows_v.at[b],
                                  sem_r.at[b]).wait()
            pltpu.make_async_copy(nd_h.at[idst_v.at[j]], nval_v.at[b],
                                  sem_n.at[b]).wait()

        def fire_scatter(j, b):
            pltpu.async_copy(rows_v.at[b], agg_sp.at[idst_v.at[j]],
                             sem_w.at[b], add=True)
            pltpu.async_copy(nval_v.at[b], s_sp.at[isrc_v.at[j]],
                             sem_x.at[b], add=True)

        def drain_scatter(j, b):
            pltpu.make_async_copy(rows_v.at[b], agg_sp.at[idst_v.at[j]],
                                  sem_w.at[b]).wait()
            pltpu.make_async_copy(nval_v.at[b], s_sp.at[isrc_v.at[j]],
                                  sem_x.at[b]).wait()

        def block(bi, _):
            pltpu.sync_copy(edge_h.at[0, wid, bi], isrc_v)
            pltpu.sync_copy(edge_h.at[1, wid, bi], idst_v)
            for j in range(LA):
                fire_gather(j, j)

            def step(j, _):
                bn = lax.rem(j + LA, NBUF)

                @pl.when(j + LA >= NBUF)
                def _():
                    drain_scatter(j + LA - NBUF, bn)

                @pl.when(j + LA < KB)
                def _():
                    fire_gather(j + LA, bn)

                b = lax.rem(j, NBUF)
                drain_gather(j, b)
                fire_scatter(j, b)
                return 0
            lax.fori_loop(0, KB, step, 0)
            for j in range(KB - (NBUF - LA), KB):
                drain_scatter(j, j % NBUF)
            return 0
        lax.fori_loop(0, nblk, block, 0)
        plsc.subcore_barrier()

        def wstep(k, _):
            off = pl.multiple_of(sid * npt + k * _CHUNK, _CHUNK)
            pltpu.sync_copy(agg_sp.at[pl.ds(off, _CHUNK)], rows_v.at[0])
            pltpu.sync_copy(rows_v.at[0], agg_h.at[cid, pl.ds(off, _CHUNK)])
            pltpu.sync_copy(s_sp.at[pl.ds(off, _CHUNK)], nval_v.at[0])
            pltpu.sync_copy(nval_v.at[0],
                            s_h.at[pl.ds(cid * NPAD + off, _CHUNK)])
            return 0
        lax.fori_loop(0, nwo, wstep, 0)

    return prop_kernel


def _make_final(NPAD, D, C, n_true, BR=2048):
    grid = NPAD // BR
    inv_n = 1.0 / float(n_true)

    nb = NPAD // BR

    def body(ap_ref, s0, s1, ns_ref, nd_ref, b1_ref, w2_ref, b2_ref,
             out_ref, acc_ref):
        i = pl.program_id(0)

        @pl.when(i == 0)
        def _():
            acc_ref[...] = jnp.zeros_like(acc_ref)

        agg = ap_ref[0] + ap_ref[1]
        h1 = jnp.maximum(agg * nd_ref[...][:, None] + b1_ref[...], 0.0)
        c = (s0[...] + s1[...]) * ns_ref[...]
        row = lax.broadcasted_iota(jnp.int32, (BR, 1), 0) + i * BR
        contrib = jnp.where(row < n_true, c[:, None] * h1, 0.0)
        acc_ref[...] += jnp.sum(contrib, axis=0, keepdims=True)

        @pl.when(i == grid - 1)
        def _():
            v = acc_ref[...]
            out_ref[...] = jnp.dot(v, w2_ref[...],
                                   preferred_element_type=jnp.float32) * inv_n \
                + b2_ref[...]

    return pl.pallas_call(
        body,
        grid=(grid,),
        in_specs=[
            pl.BlockSpec((_NC, BR, D), lambda i: (0, i, 0)),
            pl.BlockSpec((BR,), lambda i: (i,)),
            pl.BlockSpec((BR,), lambda i: (i + nb,)),
            pl.BlockSpec((BR,), lambda i: (i,)),
            pl.BlockSpec((BR,), lambda i: (i,)),
            pl.BlockSpec((1, D), lambda i: (0, 0)),
            pl.BlockSpec((D, C), lambda i: (0, 0)),
            pl.BlockSpec((1, C), lambda i: (0, 0)),
        ],
        out_specs=pl.BlockSpec((1, C), lambda i: (0, 0)),
        out_shape=jax.ShapeDtypeStruct((1, C), jnp.float32),
        scratch_shapes=[pltpu.VMEM((1, D), jnp.float32)],
    )


def kernel(x, edge_index, W1, b1, W2, b2):
    N, D = x.shape
    E = edge_index.shape[1]
    C = W2.shape[1]
    npt = -(-N // _NS)
    npt = -(-npt // _CHUNK) * _CHUNK
    NPAD = npt * _NS

    KB = 20
    ept = -(-E // _NW)
    nch = -(-ept // _CHUNK)
    nch = -(-nch // KB) * KB
    nblk = nch // KB
    E_PAD = _NW * nch * _CHUNK
    pad = jnp.full((2, E_PAD - E), N, dtype=edge_index.dtype)
    edge5 = jnp.concatenate([edge_index, pad], axis=1).reshape(
        2, _NW, nblk, KB, _CHUNK)

    dego_p, degi_p = _make_deg(E_PAD, NPAD, KB, nblk)(edge5)
    yn, ns, nd = _make_mm(NPAD, D)(x, W1, dego_p, dego_p, degi_p, degi_p)
    agg_p, s_p = _make_prop(E_PAD, NPAD, D, KB, nblk)(edge5, yn, nd)
    out = _make_final(NPAD, D, C, N)(
        agg_p, s_p, s_p, ns, nd, b1.reshape(1, D), W2, b2.reshape(1, C))
    return out


# speedup vs baseline: 2.8878x; 2.8878x over previous
import functools

import jax
import jax.numpy as jnp
from jax import lax
from jax.experimental import pallas as pl
from jax.experimental.pallas import tpu as pltpu
from jax.experimental.pallas import tpu_sc as plsc

_NC = 2
_NS = 16
_NW = _NC * _NS
_CHUNK = 80


def _mesh():
    return plsc.VectorSubcoreMesh(core_axis_name="c", subcore_axis_name="s")


def _zero_1d(ref, n):
    def f(i, _):
        ref[pl.ds(i * 16, 16)] = jnp.zeros((16,), jnp.float32)
        return 0
    lax.fori_loop(0, n // 16, f, 0)


def _zero_2d(ref, r, cdim):
    def f(i, _):
        ref[i // (cdim // 16), pl.ds((i % (cdim // 16)) * 16, 16)] = (
            jnp.zeros((16,), jnp.float32))
        return 0
    lax.fori_loop(0, r * (cdim // 16), f, 0)


def _make_deg(E, NPAD, KB, nblk):
    ept = E // _NW
    nch = ept // _CHUNK
    npt = NPAD // _NS

    @functools.partial(
        pl.kernel,
        mesh=_mesh(),
        out_type=[
            jax.ShapeDtypeStruct((_NC * NPAD,), jnp.float32),
            jax.ShapeDtypeStruct((_NC * NPAD,), jnp.float32),
        ],
        scratch_types=[
            pltpu.VMEM((nblk, KB, _CHUNK), jnp.int32),
            pltpu.VMEM((nblk, KB, _CHUNK), jnp.int32),
            pltpu.VMEM((_CHUNK,), jnp.float32),
            pltpu.VMEM((npt,), jnp.float32),
            pltpu.SemaphoreType.DMA,
            pltpu.SemaphoreType.DMA,
            pltpu.VMEM_SHARED((NPAD,), jnp.float32),
            pltpu.VMEM_SHARED((NPAD,), jnp.float32),
        ],
    )
    def deg_kernel(edge_h, dego_h, degi_h, isrc_v, idst_v, ones_v,
                   buf_v, sem_o, sem_i, dego_sp, degi_sp):
        cid = lax.axis_index("c")
        sid = lax.axis_index("s")
        wid = sid * _NC + cid

        pltpu.sync_copy(edge_h.at[0, wid], isrc_v)
        pltpu.sync_copy(edge_h.at[1, wid], idst_v)

        def fill(i, _):
            ones_v[pl.ds(i * 16, 16)] = jnp.ones((16,), jnp.float32)
            return 0
        lax.fori_loop(0, _CHUNK // 16, fill, 0)
        _zero_1d(buf_v, npt)

        pltpu.sync_copy(buf_v, dego_sp.at[pl.ds(sid * npt, npt)])
        pltpu.sync_copy(buf_v, degi_sp.at[pl.ds(sid * npt, npt)])
        plsc.subcore_barrier()

        def blk(b, _):
            def f(i, _):
                k = b * 5 + i
                pltpu.async_copy(ones_v, dego_sp.at[isrc_v.at[k // KB, k % KB]],
                                 sem_o, add=True)
                pltpu.async_copy(ones_v, degi_sp.at[idst_v.at[k // KB, k % KB]],
                                 sem_i, add=True)
                return 0
            lax.fori_loop(0, 5, f, 0)

            def d(i, _):
                k = b * 5 + i
                pltpu.make_async_copy(ones_v,
                                      dego_sp.at[isrc_v.at[k // KB, k % KB]],
                                      sem_o).wait()
                pltpu.make_async_copy(ones_v,
                                      degi_sp.at[idst_v.at[k // KB, k % KB]],
                                      sem_i).wait()
                return 0
            lax.fori_loop(0, 5, d, 0)
            return 0
        lax.fori_loop(0, nch // 5, blk, 0)

        def tail(i, _):
            pltpu.sync_copy(ones_v, dego_sp.at[isrc_v.at[i // KB, i % KB]],
                            add=True)
            pltpu.sync_copy(ones_v, degi_sp.at[idst_v.at[i // KB, i % KB]],
                            add=True)
            return 0
        lax.fori_loop((nch // 5) * 5, nch, tail, 0)
        plsc.subcore_barrier()

        pltpu.sync_copy(dego_sp.at[pl.ds(sid * npt, npt)], buf_v)
        pltpu.sync_copy(buf_v, dego_h.at[pl.ds(cid * NPAD + sid * npt, npt)])
        pltpu.sync_copy(degi_sp.at[pl.ds(sid * npt, npt)], buf_v)
        pltpu.sync_copy(buf_v, degi_h.at[pl.ds(cid * NPAD + sid * npt, npt)])

    return deg_kernel


def _make_mm(NPAD, D, BR=2048):
    def body(x_ref, w1_ref, dgo0, dgo1, dgi0, dgi1, yn_ref, ns_ref, nd_ref):
        ns = lax.rsqrt(jnp.maximum(dgo0[...] + dgo1[...], 1.0))
        nd = lax.rsqrt(jnp.maximum(dgi0[...] + dgi1[...], 1.0))
        ns_ref[...] = ns
        nd_ref[...] = nd
        yn_ref[...] = jnp.dot(x_ref[...], w1_ref[...],
                              preferred_element_type=jnp.float32) * ns[:, None]

    grid = NPAD // BR
    nb = NPAD // BR
    return pl.pallas_call(
        body,
        grid=(grid,),
        in_specs=[
            pl.BlockSpec((BR, D), lambda i: (i, 0)),
            pl.BlockSpec((D, D), lambda i: (0, 0)),
            pl.BlockSpec((BR,), lambda i: (i,)),
            pl.BlockSpec((BR,), lambda i: (i + nb,)),
            pl.BlockSpec((BR,), lambda i: (i,)),
            pl.BlockSpec((BR,), lambda i: (i + nb,)),
        ],
        out_specs=[
            pl.BlockSpec((BR, D), lambda i: (i, 0)),
            pl.BlockSpec((BR,), lambda i: (i,)),
            pl.BlockSpec((BR,), lambda i: (i,)),
        ],
        out_shape=[
            jax.ShapeDtypeStruct((NPAD, D), jnp.float32),
            jax.ShapeDtypeStruct((NPAD,), jnp.float32),
            jax.ShapeDtypeStruct((NPAD,), jnp.float32),
        ],
    )


def _make_prop(E, NPAD, D, KB, nblk):
    ept = E // _NW
    nch = ept // _CHUNK
    npt = NPAD // _NS
    nwo = npt // _CHUNK

    NBUF = 3
    LA = 1

    @functools.partial(
        pl.kernel,
        mesh=_mesh(),
        out_type=[
            jax.ShapeDtypeStruct((_NC, NPAD, D), jnp.float32),
            jax.ShapeDtypeStruct((_NC * NPAD,), jnp.float32),
        ],
        scratch_types=[
            pltpu.VMEM((KB, _CHUNK), jnp.int32),
            pltpu.VMEM((KB, _CHUNK), jnp.int32),
            pltpu.VMEM((NBUF, _CHUNK, D), jnp.float32),
            pltpu.VMEM((NBUF, _CHUNK), jnp.float32),
            pltpu.SemaphoreType.DMA((NBUF,)),
            pltpu.SemaphoreType.DMA((NBUF,)),
            pltpu.SemaphoreType.DMA((NBUF,)),
            pltpu.SemaphoreType.DMA((NBUF,)),
            pltpu.VMEM_SHARED((NPAD, D), jnp.float32),
            pltpu.VMEM_SHARED((NPAD,), jnp.float32),
        ],
    )
    def prop_kernel(edge_h, yn_h, nd_h, agg_h, s_h,
                    isrc_v, idst_v, rows_v, nval_v, sem_r, sem_n,
                    sem_w, sem_x, agg_sp, s_sp):
        cid = lax.axis_index("c")
        sid = lax.axis_index("s")
        wid = sid * _NC + cid

        _zero_2d(rows_v.at[0], _CHUNK, D)
        _zero_1d(nval_v.at[0], _CHUNK)

        def zstep(k, _):
            off = pl.multiple_of(sid * npt + k * _CHUNK, _CHUNK)
            pltpu.sync_copy(rows_v.at[0], agg_sp.at[pl.ds(off, _CHUNK)])
            pltpu.sync_copy(nval_v.at[0], s_sp.at[pl.ds(off, _CHUNK)])
            return 0
        lax.fori_loop(0, nwo, zstep, 0)
        plsc.subcore_barrier()

        def fire_gather(j, b):
            pltpu.async_copy(yn_h.at[isrc_v.at[j]], rows_v.at[b], sem_r.at[b])
            pltpu.async_copy(nd_h.at[idst_v.at[j]], nval_v.at[b], sem_n.at[b])

        def drain_gather(j, b):
            pltpu.make_async_copy(yn_h.at[isrc_v.at[j]], rows_v.at[b],
                                  sem_r.at[b]).wait()
            pltpu.make_async_copy(nd_h.at[idst_v.at[j]], nval_v.at[b],
                                  sem_n.at[b]).wait()

        def fire_scatter(j, b):
            pltpu.async_copy(rows_v.at[b], agg_sp.at[idst_v.at[j]],
                             sem_w.at[b], add=True)
            pltpu.async_copy(nval_v.at[b], s_sp.at[isrc_v.at[j]],
                             sem_x.at[b], add=True)

        def drain_scatter(j, b):
            pltpu.make_async_copy(rows_v.at[b], agg_sp.at[idst_v.at[j]],
                                  sem_w.at[b]).wait()
            pltpu.make_async_copy(nval_v.at[b], s_sp.at[isrc_v.at[j]],
                                  sem_x.at[b]).wait()

        def block(bi, _):
            pltpu.sync_copy(edge_h.at[0, wid, bi], isrc_v)
            pltpu.sync_copy(edge_h.at[1, wid, bi], idst_v)
            for j in range(LA):
                fire_gather(j, j)

            def step(j, _):
                bn = lax.rem(j + LA, NBUF)

                @pl.when(j + LA >= NBUF)
                def _():
                    drain_scatter(j + LA - NBUF, bn)

                @pl.when(j + LA < KB)
                def _():
                    fire_gather(j + LA, bn)

                b = lax.rem(j, NBUF)
                drain_gather(j, b)
                fire_scatter(j, b)
                return 0
            lax.fori_loop(0, KB, step, 0)
            for j in range(KB - (NBUF - LA), KB):
                drain_scatter(j, j % NBUF)
            return 0
        lax.fori_loop(0, nblk, block, 0)
        plsc.subcore_barrier()

        def wstep(k, _):
            off = pl.multiple_of(sid * npt + k * _CHUNK, _CHUNK)
            pltpu.sync_copy(agg_sp.at[pl.ds(off, _CHUNK)], rows_v.at[0])
            pltpu.sync_copy(rows_v.at[0], agg_h.at[cid, pl.ds(off, _CHUNK)])
            pltpu.sync_copy(s_sp.at[pl.ds(off, _CHUNK)], nval_v.at[0])
            pltpu.sync_copy(nval_v.at[0],
                            s_h.at[pl.ds(cid * NPAD + off, _CHUNK)])
            return 0
        lax.fori_loop(0, nwo, wstep, 0)

    return prop_kernel


def _make_final(NPAD, D, C, n_true, BR=2048):
    grid = NPAD // BR
    inv_n = 1.0 / float(n_true)

    nb = NPAD // BR

    def body(ap_ref, s0, s1, ns_ref, nd_ref, b1_ref, w2_ref, b2_ref,
             out_ref, acc_ref):
        i = pl.program_id(0)

        @pl.when(i == 0)
        def _():
            acc_ref[...] = jnp.zeros_like(acc_ref)

        agg = ap_ref[0] + ap_ref[1]
        h1 = jnp.maximum(agg * nd_ref[...][:, None] + b1_ref[...], 0.0)
        c = (s0[...] + s1[...]) * ns_ref[...]
        acc_ref[...] += jnp.sum(c[:, None] * h1, axis=0, keepdims=True)

        @pl.when(i == grid - 1)
        def _():
            v = acc_ref[...]
            out_ref[...] = jnp.dot(v, w2_ref[...],
                                   preferred_element_type=jnp.float32) * inv_n \
                + b2_ref[...]

    return pl.pallas_call(
        body,
        grid=(grid,),
        in_specs=[
            pl.BlockSpec((_NC, BR, D), lambda i: (0, i, 0)),
            pl.BlockSpec((BR,), lambda i: (i,)),
            pl.BlockSpec((BR,), lambda i: (i + nb,)),
            pl.BlockSpec((BR,), lambda i: (i,)),
            pl.BlockSpec((BR,), lambda i: (i,)),
            pl.BlockSpec((1, D), lambda i: (0, 0)),
            pl.BlockSpec((D, C), lambda i: (0, 0)),
            pl.BlockSpec((1, C), lambda i: (0, 0)),
        ],
        out_specs=pl.BlockSpec((1, C), lambda i: (0, 0)),
        out_shape=jax.ShapeDtypeStruct((1, C), jnp.float32),
        scratch_shapes=[pltpu.VMEM((1, D), jnp.float32)],
    )


def kernel(x, edge_index, W1, b1, W2, b2):
    N, D = x.shape
    E = edge_index.shape[1]
    C = W2.shape[1]
    npt = -(-N // _NS)
    npt = -(-npt // _CHUNK) * _CHUNK
    NPAD = npt * _NS

    ept = E // _NW
    nch = ept // _CHUNK
    KB = 25
    nblk = nch // KB
    edge5 = edge_index.reshape(2, _NW, nblk, KB, _CHUNK)

    dego_p, degi_p = _make_deg(E, NPAD, KB, nblk)(edge5)
    yn, ns, nd = _make_mm(NPAD, D)(x, W1, dego_p, dego_p, degi_p, degi_p)
    agg_p, s_p = _make_prop(E, NPAD, D, KB, nblk)(edge5, yn, nd)
    out = _make_final(NPAD, D, C, N)(
        agg_p, s_p, s_p, ns, nd, b1.reshape(1, D), W2, b2.reshape(1, C))
    return out


# speedup vs baseline: 2.9593x; 1.0248x over previous
import functools

import jax
import jax.numpy as jnp
from jax import lax
from jax.experimental import pallas as pl
from jax.experimental.pallas import tpu as pltpu
from jax.experimental.pallas import tpu_sc as plsc

_NC = 2
_NS = 16
_NW = _NC * _NS
_CHUNK = 80


def _mesh():
    return plsc.VectorSubcoreMesh(core_axis_name="c", subcore_axis_name="s")


def _zero_1d(ref, n):
    def f(i, _):
        ref[pl.ds(i * 16, 16)] = jnp.zeros((16,), jnp.float32)
        return 0
    lax.fori_loop(0, n // 16, f, 0)


def _zero_2d(ref, r, cdim):
    def f(i, _):
        ref[i // (cdim // 16), pl.ds((i % (cdim // 16)) * 16, 16)] = (
            jnp.zeros((16,), jnp.float32))
        return 0
    lax.fori_loop(0, r * (cdim // 16), f, 0)


def _make_deg(E, NPAD):
    CH = 128
    nch_all = E // CH
    nch = nch_all // _NW
    tail = nch_all % _NW
    npt = NPAD // _NS
    NB = 8
    LI = 4

    @functools.partial(
        pl.kernel,
        mesh=_mesh(),
        out_type=[
            jax.ShapeDtypeStruct((_NC * NPAD,), jnp.float32),
            jax.ShapeDtypeStruct((_NC * NPAD,), jnp.float32),
        ],
        scratch_types=[
            pltpu.VMEM((NB, 2, CH), jnp.int32),
            pltpu.VMEM((CH,), jnp.float32),
            pltpu.VMEM((npt,), jnp.float32),
            pltpu.SemaphoreType.DMA((NB,)),
            pltpu.SemaphoreType.DMA((NB,)),
            pltpu.SemaphoreType.DMA((NB,)),
            pltpu.VMEM_SHARED((NPAD,), jnp.float32),
            pltpu.VMEM_SHARED((NPAD,), jnp.float32),
        ],
    )
    def deg_kernel(edge_h, dego_h, degi_h, ibuf_v, ones_v,
                   buf_v, sem_e, sem_o, sem_i, dego_sp, degi_sp):
        cid = lax.axis_index("c")
        sid = lax.axis_index("s")
        wid = sid * _NC + cid
        base = wid * nch * CH

        def fill(i, _):
            ones_v[pl.ds(i * 16, 16)] = jnp.ones((16,), jnp.float32)
            return 0
        lax.fori_loop(0, CH // 16, fill, 0)
        _zero_1d(buf_v, npt)

        pltpu.sync_copy(buf_v, dego_sp.at[pl.ds(sid * npt, npt)])
        pltpu.sync_copy(buf_v, degi_sp.at[pl.ds(sid * npt, npt)])
        plsc.subcore_barrier()

        def fire_idx(j, b):
            pltpu.async_copy(edge_h.at[:, pl.ds(base + j * CH, CH)],
                             ibuf_v.at[b], sem_e.at[b])

        def drain_idx(j, b):
            pltpu.make_async_copy(edge_h.at[:, pl.ds(base + j * CH, CH)],
                                  ibuf_v.at[b], sem_e.at[b]).wait()

        def fire_sc(j, b):
            pltpu.async_copy(ones_v, dego_sp.at[ibuf_v.at[b, 0]],
                             sem_o.at[b], add=True)
            pltpu.async_copy(ones_v, degi_sp.at[ibuf_v.at[b, 1]],
                             sem_i.at[b], add=True)

        def drain_sc(j, b):
            pltpu.make_async_copy(ones_v, dego_sp.at[ibuf_v.at[b, 0]],
                                  sem_o.at[b]).wait()
            pltpu.make_async_copy(ones_v, degi_sp.at[ibuf_v.at[b, 1]],
                                  sem_i.at[b]).wait()

        for j in range(LI):
            fire_idx(j, j)

        def step(j, _):
            bn = lax.rem(j + LI, NB)

            @pl.when(j + LI >= NB)
            def _():
                drain_sc(j + LI - NB, bn)

            @pl.when(j + LI < nch)
            def _():
                fire_idx(j + LI, bn)

            b = lax.rem(j, NB)
            drain_idx(j, b)
            fire_sc(j, b)
            return 0
        lax.fori_loop(0, nch, step, 0)
        for j in range(max(0, nch - (NB - LI)), nch):
            drain_sc(j, j % NB)

        @pl.when(wid < tail)
        def _():
            off = (nch * _NW + wid) * CH
            pltpu.sync_copy(edge_h.at[:, pl.ds(off, CH)], ibuf_v.at[0])
            pltpu.sync_copy(ones_v, dego_sp.at[ibuf_v.at[0, 0]], add=True)
            pltpu.sync_copy(ones_v, degi_sp.at[ibuf_v.at[0, 1]], add=True)
        plsc.subcore_barrier()

        pltpu.sync_copy(dego_sp.at[pl.ds(sid * npt, npt)], buf_v)
        pltpu.sync_copy(buf_v, dego_h.at[pl.ds(cid * NPAD + sid * npt, npt)])
        pltpu.sync_copy(degi_sp.at[pl.ds(sid * npt, npt)], buf_v)
        pltpu.sync_copy(buf_v, degi_h.at[pl.ds(cid * NPAD + sid * npt, npt)])

    return deg_kernel


def _make_mm(NPAD, D, BR=2048):
    def body(x_ref, w1_ref, dgo0, dgo1, dgi0, dgi1, yn_ref, ns_ref, nd_ref):
        ns = lax.rsqrt(jnp.maximum(dgo0[...] + dgo1[...], 1.0))
        nd = lax.rsqrt(jnp.maximum(dgi0[...] + dgi1[...], 1.0))
        ns_ref[...] = ns
        nd_ref[...] = nd
        yn_ref[...] = jnp.dot(x_ref[...], w1_ref[...],
                              preferred_element_type=jnp.float32) * ns[:, None]

    grid = NPAD // BR
    nb = NPAD // BR
    return pl.pallas_call(
        body,
        grid=(grid,),
        in_specs=[
            pl.BlockSpec((BR, D), lambda i: (i, 0)),
            pl.BlockSpec((D, D), lambda i: (0, 0)),
            pl.BlockSpec((BR,), lambda i: (i,)),
            pl.BlockSpec((BR,), lambda i: (i + nb,)),
            pl.BlockSpec((BR,), lambda i: (i,)),
            pl.BlockSpec((BR,), lambda i: (i + nb,)),
        ],
        out_specs=[
            pl.BlockSpec((BR, D), lambda i: (i, 0)),
            pl.BlockSpec((BR,), lambda i: (i,)),
            pl.BlockSpec((BR,), lambda i: (i,)),
        ],
        out_shape=[
            jax.ShapeDtypeStruct((NPAD, D), jnp.float32),
            jax.ShapeDtypeStruct((NPAD,), jnp.float32),
            jax.ShapeDtypeStruct((NPAD,), jnp.float32),
        ],
    )


def _make_prop(E, NPAD, D, KB, nblk):
    ept = E // _NW
    nch = ept // _CHUNK
    npt = NPAD // _NS
    nwo = npt // _CHUNK

    NBUF = 3
    LA = 1

    @functools.partial(
        pl.kernel,
        mesh=_mesh(),
        out_type=[
            jax.ShapeDtypeStruct((_NC, NPAD, D), jnp.float32),
            jax.ShapeDtypeStruct((_NC * NPAD,), jnp.float32),
        ],
        scratch_types=[
            pltpu.VMEM((KB, _CHUNK), jnp.int32),
            pltpu.VMEM((KB, _CHUNK), jnp.int32),
            pltpu.VMEM((NBUF, _CHUNK, D), jnp.float32),
            pltpu.VMEM((NBUF, _CHUNK), jnp.float32),
            pltpu.SemaphoreType.DMA((NBUF,)),
            pltpu.SemaphoreType.DMA((NBUF,)),
            pltpu.SemaphoreType.DMA((NBUF,)),
            pltpu.SemaphoreType.DMA((NBUF,)),
            pltpu.VMEM_SHARED((NPAD, D), jnp.float32),
            pltpu.VMEM_SHARED((NPAD,), jnp.float32),
        ],
    )
    def prop_kernel(edge_h, yn_h, nd_h, agg_h, s_h,
                    isrc_v, idst_v, rows_v, nval_v, sem_r, sem_n,
                    sem_w, sem_x, agg_sp, s_sp):
        cid = lax.axis_index("c")
        sid = lax.axis_index("s")
        wid = sid * _NC + cid

        _zero_2d(rows_v.at[0], _CHUNK, D)
        _zero_1d(nval_v.at[0], _CHUNK)

        def zstep(k, _):
            off = pl.multiple_of(sid * npt + k * _CHUNK, _CHUNK)
            pltpu.sync_copy(rows_v.at[0], agg_sp.at[pl.ds(off, _CHUNK)])
            pltpu.sync_copy(nval_v.at[0], s_sp.at[pl.ds(off, _CHUNK)])
            return 0
        lax.fori_loop(0, nwo, zstep, 0)
        plsc.subcore_barrier()

        def fire_gather(j, b):
            pltpu.async_copy(yn_h.at[isrc_v.at[j]], rows_v.at[b], sem_r.at[b])
            pltpu.async_copy(nd_h.at[idst_v.at[j]], nval_v.at[b], sem_n.at[b])

        def drain_gather(j, b):
            pltpu.make_async_copy(yn_h.at[isrc_v.at[j]], rows_v.at[b],
                                  sem_r.at[b]).wait()
            pltpu.make_async_copy(nd_h.at[idst_v.at[j]], nval_v.at[b],
                                  sem_n.at[b]).wait()

        def fire_scatter(j, b):
            pltpu.async_copy(rows_v.at[b], agg_sp.at[idst_v.at[j]],
                             sem_w.at[b], add=True)
            pltpu.async_copy(nval_v.at[b], s_sp.at[isrc_v.at[j]],
                             sem_x.at[b], add=True)

        def drain_scatter(j, b):
            pltpu.make_async_copy(rows_v.at[b], agg_sp.at[idst_v.at[j]],
                                  sem_w.at[b]).wait()
            pltpu.make_async_copy(nval_v.at[b], s_sp.at[isrc_v.at[j]],
                                  sem_x.at[b]).wait()

        def block(bi, _):
            pltpu.sync_copy(edge_h.at[0, wid, bi], isrc_v)
            pltpu.sync_copy(edge_h.at[1, wid, bi], idst_v)
            for j in range(LA):
                fire_gather(j, j)

            def step(j, _):
                bn = lax.rem(j + LA, NBUF)

                @pl.when(j + LA >= NBUF)
                def _():
                    drain_scatter(j + LA - NBUF, bn)

                @pl.when(j + LA < KB)
                def _():
                    fire_gather(j + LA, bn)

                b = lax.rem(j, NBUF)
                drain_gather(j, b)
                fire_scatter(j, b)
                return 0
            lax.fori_loop(0, KB, step, 0)
            for j in range(KB - (NBUF - LA), KB):
                drain_scatter(j, j % NBUF)
            return 0
        lax.fori_loop(0, nblk, block, 0)
        plsc.subcore_barrier()

        def wstep(k, _):
            off = pl.multiple_of(sid * npt + k * _CHUNK, _CHUNK)
            pltpu.sync_copy(agg_sp.at[pl.ds(off, _CHUNK)], rows_v.at[0])
            pltpu.sync_copy(rows_v.at[0], agg_h.at[cid, pl.ds(off, _CHUNK)])
            pltpu.sync_copy(s_sp.at[pl.ds(off, _CHUNK)], nval_v.at[0])
            pltpu.sync_copy(nval_v.at[0],
                            s_h.at[pl.ds(cid * NPAD + off, _CHUNK)])
            return 0
        lax.fori_loop(0, nwo, wstep, 0)

    return prop_kernel


def _make_final(NPAD, D, C, n_true, BR=2048):
    grid = NPAD // BR
    inv_n = 1.0 / float(n_true)

    nb = NPAD // BR

    def body(ap_ref, s0, s1, ns_ref, nd_ref, b1_ref, w2_ref, b2_ref,
             out_ref, acc_ref):
        i = pl.program_id(0)

        @pl.when(i == 0)
        def _():
            acc_ref[...] = jnp.zeros_like(acc_ref)

        agg = ap_ref[0] + ap_ref[1]
        h1 = jnp.maximum(agg * nd_ref[...][:, None] + b1_ref[...], 0.0)
        c = (s0[...] + s1[...]) * ns_ref[...]
        acc_ref[...] += jnp.sum(c[:, None] * h1, axis=0, keepdims=True)

        @pl.when(i == grid - 1)
        def _():
            v = acc_ref[...]
            out_ref[...] = jnp.dot(v, w2_ref[...],
                                   preferred_element_type=jnp.float32) * inv_n \
                + b2_ref[...]

    return pl.pallas_call(
        body,
        grid=(grid,),
        in_specs=[
            pl.BlockSpec((_NC, BR, D), lambda i: (0, i, 0)),
            pl.BlockSpec((BR,), lambda i: (i,)),
            pl.BlockSpec((BR,), lambda i: (i + nb,)),
            pl.BlockSpec((BR,), lambda i: (i,)),
            pl.BlockSpec((BR,), lambda i: (i,)),
            pl.BlockSpec((1, D), lambda i: (0, 0)),
            pl.BlockSpec((D, C), lambda i: (0, 0)),
            pl.BlockSpec((1, C), lambda i: (0, 0)),
        ],
        out_specs=pl.BlockSpec((1, C), lambda i: (0, 0)),
        out_shape=jax.ShapeDtypeStruct((1, C), jnp.float32),
        scratch_shapes=[pltpu.VMEM((1, D), jnp.float32)],
    )


def kernel(x, edge_index, W1, b1, W2, b2):
    N, D = x.shape
    E = edge_index.shape[1]
    C = W2.shape[1]
    npt = -(-N // _NS)
    npt = -(-npt // _CHUNK) * _CHUNK
    NPAD = npt * _NS

    ept = E // _NW
    nch = ept // _CHUNK
    KB = 25
    nblk = nch // KB
    edge5 = edge_index.reshape(2, _NW, nblk, KB, _CHUNK)

    dego_p, degi_p = _make_deg(E, NPAD)(edge_index)
    yn, ns, nd = _make_mm(NPAD, D)(x, W1, dego_p, dego_p, degi_p, degi_p)
    agg_p, s_p = _make_prop(E, NPAD, D, KB, nblk)(edge5, yn, nd)
    out = _make_final(NPAD, D, C, N)(
        agg_p, s_p, s_p, ns, nd, b1.reshape(1, D), W2, b2.reshape(1, C))
    return out
